# Initial kernel scaffold; baseline (speedup 1.0000x reference)
import jax
import jax.numpy as jnp
from jax.experimental import pallas as pl


def kernel(one_hot_ids, multi_hot_ids, dense_feats, one_hot_tables, multi_hot_table, W, b):
    raise NotImplementedError("write your pallas kernel here")


# speedup vs baseline: 2.0390x; 2.0390x over previous
import functools

import jax
import jax.numpy as jnp
from jax import lax
from jax.experimental import pallas as pl
from jax.experimental.pallas import tpu as pltpu
from jax.experimental.pallas import tpu_sc as plsc

NC = 2
NS = 16
NW = NC * NS


@functools.partial(jax.jit, static_argnames=("interpret",))
def _sc_lr(oh_idsT, mh_idsT, dense_pad, oh_tab, mh_tab, w_all, interpret=False):
    F, B = oh_idsT.shape
    L = mh_idsT.shape[0]
    V, D = mh_tab.shape
    SPW = B // NW
    NG = SPW // 16
    mesh = plsc.VectorSubcoreMesh(core_axis_name="c", subcore_axis_name="s",
                                  num_cores=NC, num_subcores=NS)

    @functools.partial(
        pl.kernel,
        out_type=jax.ShapeDtypeStruct((B,), jnp.float32),
        mesh=mesh,
        interpret=interpret,
        compiler_params=pltpu.CompilerParams(use_tc_tiling_on_sc=False),
        scratch_types=[
            pltpu.VMEM((F, SPW), jnp.int32),
            pltpu.VMEM((L, SPW), jnp.int32),
            pltpu.VMEM((SPW, D), jnp.float32),
            pltpu.VMEM((32, D), jnp.float32),
            pltpu.VMEM((SPW * D,), jnp.float32),
            pltpu.VMEM((SPW, D), jnp.float32),
            pltpu.VMEM((SPW,), jnp.float32),
            pltpu.SemaphoreType.DMA,
        ],
    )
    def k(oh_idsT_h, mh_idsT_h, dense_h, oh_tab_h, mh_tab_h, w_h, out_h,
          oh_idx, mh_idx, dense_v, w_v, acc, rows, out_v, sem):
        wid = lax.axis_index("s") * NC + lax.axis_index("c")
        base = wid * SPW

        pltpu.sync_copy(oh_idsT_h.at[:, pl.ds(base, SPW)], oh_idx)
        pltpu.sync_copy(mh_idsT_h.at[:, pl.ds(base, SPW)], mh_idx)
        pltpu.sync_copy(dense_h.at[pl.ds(base, SPW)], dense_v)
        pltpu.sync_copy(w_h, w_v)

        def offs_f(f, _):
            off = f * V

            def offs_i(i, _):
                sl = pl.ds(i * 16, 16)
                oh_idx[f, sl] = oh_idx[f, sl] + off
                return 0

            return lax.fori_loop(0, SPW // 16, offs_i, 0)

        lax.fori_loop(0, F, offs_f, 0)

        wd = w_v[F + 1]
        brow = w_v[F + 2]

        def init_s(s, _):
            acc[pl.ds(s * D, D)] = dense_v[s] * wd + brow
            return 0

        lax.fori_loop(0, SPW, init_s, 0)

        def oh_f(f, _):
            pltpu.async_copy(oh_tab_h.at[oh_idx.at[f]], rows, sem).wait()
            wf = w_v[f]

            def acc_s(s, _):
                sl = pl.ds(s * D, D)
                acc[sl] = acc[sl] + rows[s] * wf
                return 0

            return lax.fori_loop(0, SPW, acc_s, 0)

        lax.fori_loop(0, F, oh_f, 0)

        wm = w_v[F] * (1.0 / L)

        def mh_l(l, _):
            pltpu.async_copy(mh_tab_h.at[mh_idx.at[l]], rows, sem).wait()

            def acc_s(s, _):
                sl = pl.ds(s * D, D)
                acc[sl] = acc[sl] + rows[s] * wm
                return 0

            return lax.fori_loop(0, SPW, acc_s, 0)

        lax.fori_loop(0, L, mh_l, 0)

        lane = lax.iota(jnp.int32, 16)

        def red_g(g, _):
            def red_s(j, tv):
                v = acc[pl.ds((g * 16 + j) * D, D)]
                for sh in (8, 4, 2, 1):
                    v = v + lax.gather(
                        v, (lane ^ sh)[:, None],
                        lax.GatherDimensionNumbers(
                            offset_dims=(), collapsed_slice_dims=(0,),
                            start_index_map=(0,)),
                        slice_sizes=(1,),
                        mode=lax.GatherScatterMode.PROMISE_IN_BOUNDS)
                return jnp.where(lane == j, v, tv)

            tv = lax.fori_loop(0, 16, red_s, jnp.zeros((16,), jnp.float32))
            out_v[pl.ds(g * 16, 16)] = 1.0 / (1.0 + jnp.exp(-tv))
            return 0

        lax.fori_loop(0, NG, red_g, 0)
        pltpu.sync_copy(out_v, out_h.at[pl.ds(base, SPW)])

    return k(oh_idsT, mh_idsT, dense_pad, oh_tab, mh_tab, w_all)


def kernel(one_hot_ids, multi_hot_ids, dense_feats, one_hot_tables,
           multi_hot_table, W, b):
    B, F = one_hot_ids.shape
    V, D = multi_hot_table.shape
    DD = dense_feats.shape[1]
    oh_idsT = one_hot_ids.T
    mh_idsT = multi_hot_ids.T
    dense_pad = jnp.pad(dense_feats, ((0, 0), (0, D - DD)))
    w = W[:, 0]
    w_oh = w[: F * D].reshape(F, D)
    w_mh = w[F * D: F * D + D].reshape(1, D)
    w_dn = jnp.pad(w[F * D + D:], (0, D - DD)).reshape(1, D)
    brow = jnp.pad(b.astype(jnp.float32), (0, D - 1)).reshape(1, D)
    w_all = jnp.concatenate(
        [w_oh, w_mh, w_dn, brow, jnp.zeros((32 - F - 3, D), jnp.float32)], 0)
    out = _sc_lr(oh_idsT, mh_idsT, dense_pad, one_hot_tables,
                 multi_hot_table, w_all)
    return out.reshape(B, 1)


# speedup vs baseline: 2.1852x; 1.0717x over previous
import functools

import jax
import jax.numpy as jnp
from jax import lax
from jax.experimental import pallas as pl
from jax.experimental.pallas import tpu as pltpu
from jax.experimental.pallas import tpu_sc as plsc

NC = 2
NS = 16
NW = NC * NS
CH = 32


def _lane_sum(v, lane):
    for sh in (8, 4, 2, 1):
        v = v + lax.gather(
            v, (lane ^ sh)[:, None],
            lax.GatherDimensionNumbers(
                offset_dims=(), collapsed_slice_dims=(0,),
                start_index_map=(0,)),
            slice_sizes=(1,),
            mode=lax.GatherScatterMode.PROMISE_IN_BOUNDS)
    return v


@functools.partial(jax.jit, static_argnames=("interpret",))
def _sc_lr(oh_idsT, mh_idsT, dense_pad, oh_tab, mh_tab, w_all, interpret=False):
    F, B = oh_idsT.shape
    L = mh_idsT.shape[0]
    V, D = mh_tab.shape
    NF = F + L
    SPW = B // NW
    NCH = SPW // CH
    mesh = plsc.VectorSubcoreMesh(core_axis_name="c", subcore_axis_name="s",
                                  num_cores=NC, num_subcores=NS)

    @functools.partial(
        pl.kernel,
        out_type=jax.ShapeDtypeStruct((B,), jnp.float32),
        mesh=mesh,
        interpret=interpret,
        compiler_params=pltpu.CompilerParams(use_tc_tiling_on_sc=False),
        scratch_types=[
            pltpu.VMEM((NF, SPW), jnp.int32),
            pltpu.VMEM((2, NF, CH, D), jnp.float32),
            pltpu.VMEM((SPW, D), jnp.float32),
            pltpu.VMEM((32, D), jnp.float32),
            pltpu.VMEM((SPW,), jnp.float32),
            pltpu.SemaphoreType.DMA,
            pltpu.SemaphoreType.DMA,
        ],
    )
    def k(oh_idsT_h, mh_idsT_h, dense_h, oh_tab_h, mh_tab_h, w_h, out_h,
          idx_all, rows2, dense_v, w_v, out_v, sem0, sem1):
        wid = lax.axis_index("s") * NC + lax.axis_index("c")
        base = wid * SPW
        sems = (sem0, sem1)

        pltpu.sync_copy(oh_idsT_h.at[:, pl.ds(base, SPW)],
                        idx_all.at[pl.ds(0, F)])
        pltpu.sync_copy(mh_idsT_h.at[:, pl.ds(base, SPW)],
                        idx_all.at[pl.ds(F, L)])
        pltpu.sync_copy(dense_h.at[pl.ds(base, SPW)], dense_v)
        pltpu.sync_copy(w_h, w_v)

        for f in range(F):
            def offs_i(i, _, f=f):
                sl = pl.ds(i * 16, 16)
                idx_all[f, sl] = idx_all[f, sl] + (f * V)
                return 0
            lax.fori_loop(0, SPW // 16, offs_i, 0)

        def fire(c):
            buf = c & 1
            descs = []
            for j in range(NF):
                tab = oh_tab_h if j < F else mh_tab_h
                descs.append(pltpu.async_copy(
                    tab.at[idx_all.at[j, pl.ds(c * CH, CH)]],
                    rows2.at[buf, j], sems[buf]))
            return descs

        wd = w_v[F + 1]
        brow = w_v[F + 2]
        wm = w_v[F] * (1.0 / L)
        ws = [w_v[f] for f in range(F)]
        lane = lax.iota(jnp.int32, 16)

        def compute(c):
            buf = c & 1

            def body(s, tv):
                acc = dense_v[c * CH + s] * wd + brow
                for f in range(F):
                    acc = acc + rows2[buf, f, s] * ws[f]
                m = [rows2[buf, F + t, s] for t in range(4)]
                for l in range(4, L):
                    m[l & 3] = m[l & 3] + rows2[buf, F + l, s]
                acc = acc + ((m[0] + m[1]) + (m[2] + m[3])) * wm
                tv = jnp.where(lane == (s & 15), _lane_sum(acc, lane), tv)

                @pl.when((s & 15) == 15)
                def _():
                    out_v[pl.ds(c * CH + ((s >> 4) << 4), 16)] = (
                        1.0 / (1.0 + jnp.exp(-tv)))

                return tv

            lax.fori_loop(0, CH, body, jnp.zeros((16,), jnp.float32))

        descs = {0: fire(0), 1: fire(1)}
        for c in range(NCH):
            for d in descs.pop(c):
                d.wait()
            compute(c)
            if c + 2 < NCH:
                descs[c + 2] = fire(c + 2)

        pltpu.sync_copy(out_v, out_h.at[pl.ds(base, SPW)])

    return k(oh_idsT, mh_idsT, dense_pad, oh_tab, mh_tab, w_all)


def kernel(one_hot_ids, multi_hot_ids, dense_feats, one_hot_tables,
           multi_hot_table, W, b):
    B, F = one_hot_ids.shape
    V, D = multi_hot_table.shape
    DD = dense_feats.shape[1]
    oh_idsT = one_hot_ids.T
    mh_idsT = multi_hot_ids.T
    dense_pad = jnp.pad(dense_feats, ((0, 0), (0, D - DD)))
    w = W[:, 0]
    w_oh = w[: F * D].reshape(F, D)
    w_mh = w[F * D: F * D + D].reshape(1, D)
    w_dn = jnp.pad(w[F * D + D:], (0, D - DD)).reshape(1, D)
    brow = jnp.pad(b.astype(jnp.float32), (0, D - 1)).reshape(1, D)
    w_all = jnp.concatenate(
        [w_oh, w_mh, w_dn, brow, jnp.zeros((32 - F - 3, D), jnp.float32)], 0)
    out = _sc_lr(oh_idsT, mh_idsT, dense_pad, one_hot_tables,
                 multi_hot_table, w_all)
    return out.reshape(B, 1)
